# baseline (speedup 1.0000x reference)
import functools

import jax
import jax.numpy as jnp
from jax import lax
from jax.experimental import pallas as pl
from jax.experimental.pallas import tpu as pltpu
from jax.experimental.pallas import tpu_sc as plsc

N = 10000
NP = 10240
E = 320000
G = 256
D = 128

_KC = 64
_NSUB = 16
_NCORE = 2
_NCH = 160
_EPT = _NCH * _KC
_EPAD = _EPT * _NCORE * _NSUB
_RPT = NP // _NSUB


@functools.lru_cache(maxsize=None)
def _sc_kernels():
    mesh = plsc.VectorSubcoreMesh(core_axis_name="c", subcore_axis_name="s")

    @functools.partial(
        pl.kernel,
        mesh=mesh,
        out_type=jax.ShapeDtypeStruct((_NCORE, NP), jnp.float32),
        scratch_types=[
            pltpu.VMEM((_NCH, _KC), jnp.int32),
            pltpu.VMEM((_KC,), jnp.float32),
            pltpu.VMEM_SHARED((NP,), jnp.float32),
            pltpu.SemaphoreType.DMA,
        ],
    )
    def sc_degree(dst_hbm, zeros_hbm, out_hbm, di_v, ones_v, acc, sem):
        c = lax.axis_index("c")
        s = lax.axis_index("s")
        for i in range(_KC // 16):
            ones_v[pl.ds(i * 16, 16)] = jnp.ones((16,), jnp.float32)

        row0 = (c * _NSUB + s) * _NCH
        pltpu.sync_copy(dst_hbm.at[pl.ds(row0, _NCH)], di_v)
        pltpu.sync_copy(zeros_hbm.at[pl.ds(s * _RPT, _RPT)],
                        acc.at[pl.ds(s * _RPT, _RPT)])
        plsc.subcore_barrier()

        def body(k, carry):
            j = pl.multiple_of(k * 8, 8)
            for u in range(8):
                pltpu.async_copy(ones_v, acc.at[di_v.at[j + u]], sem, add=True)
            for u in range(8):
                pltpu.make_async_copy(ones_v, acc.at[di_v.at[j + u]], sem).wait()
            return carry

        lax.fori_loop(0, _NCH // 8, body, 0)
        plsc.subcore_barrier()
        pltpu.sync_copy(acc.at[pl.ds(s * _RPT, _RPT)],
                        out_hbm.at[c, pl.ds(s * _RPT, _RPT)])

    @functools.partial(
        pl.kernel,
        mesh=mesh,
        out_type=jax.ShapeDtypeStruct((_NCORE, NP, D), jnp.float32),
        scratch_types=[
            pltpu.VMEM((_KC, D), jnp.float32),
            pltpu.VMEM((_KC, D), jnp.float32),
            pltpu.VMEM((_KC, D), jnp.float32),
            pltpu.VMEM((_KC, D), jnp.float32),
            pltpu.VMEM((8, _KC), jnp.int32),
            pltpu.VMEM((8, _KC), jnp.int32),
            pltpu.SemaphoreType.DMA,
            pltpu.SemaphoreType.DMA,
            pltpu.SemaphoreType.DMA,
            pltpu.SemaphoreType.DMA,
            pltpu.SemaphoreType.DMA,
            pltpu.SemaphoreType.DMA,
            pltpu.SemaphoreType.DMA,
            pltpu.SemaphoreType.DMA,
            pltpu.SemaphoreType.DMA,
            pltpu.SemaphoreType.DMA,
            pltpu.SemaphoreType.DMA,
            pltpu.SemaphoreType.DMA,
            pltpu.SemaphoreType.DMA,
            pltpu.SemaphoreType.DMA,
            pltpu.SemaphoreType.DMA,
            pltpu.SemaphoreType.DMA,
            pltpu.VMEM_SHARED((NP, D), jnp.float32),
        ],
    )
    def sc_propagate(table_hbm, src_hbm, dst_hbm, zeros_hbm, out_hbm,
                     r0, r1, r2, r3, sb, db,
                     i0, i1, i2, i3, i4, i5, i6, i7, g0, g1, g2, g3,
                     t0, t1, t2, t3, acc):
        c = lax.axis_index("c")
        s = lax.axis_index("s")
        rows = (r0, r1, r2, r3)
        isem = (i0, i1, i2, i3, i4, i5, i6, i7)
        gsem = (g0, g1, g2, g3)
        ssem = (t0, t1, t2, t3)

        base = (c * _NSUB + s) * _EPT

        def idx_load(j, u6):
            off = pl.multiple_of(base + j * _KC, 8)
            pltpu.async_copy(src_hbm.at[pl.ds(off, _KC)], sb.at[u6], isem[u6])
            pltpu.async_copy(dst_hbm.at[pl.ds(off, _KC)], db.at[u6], isem[u6])

        def idx_wait(u6):
            pltpu.make_async_copy(src_hbm.at[pl.ds(0, _KC)], sb.at[u6],
                                  isem[u6]).wait()
            pltpu.make_async_copy(dst_hbm.at[pl.ds(0, _KC)], db.at[u6],
                                  isem[u6]).wait()

        for m in range(7):
            idx_load(m, m)
        for m in range(3):
            idx_wait(m)
            pltpu.async_copy(table_hbm.at[sb.at[m]], rows[m], gsem[m])
        pltpu.sync_copy(zeros_hbm.at[pl.ds(s * _RPT, _RPT)],
                        acc.at[pl.ds(s * _RPT, _RPT)])
        plsc.subcore_barrier()

        def scat_wait(jm1, u4):
            pltpu.make_async_copy(rows[u4], acc.at[db.at[jm1 % 8]],
                                  ssem[u4]).wait()

        def step(j, u4, u8, first=False):
            @pl.when(j + 3 < _NCH)
            def _launch():
                idx_wait((u8 + 3) % 8)
                if not first:
                    scat_wait(j - 1, (u4 + 3) % 4)
                pltpu.async_copy(table_hbm.at[sb.at[(u8 + 3) % 8]],
                                 rows[(u4 + 3) % 4], gsem[(u4 + 3) % 4])

                @pl.when(j + 7 < _NCH)
                def _next_idx():
                    idx_load(j + 7, (u8 + 7) % 8)

            pltpu.make_async_copy(table_hbm.at[sb.at[u8]], rows[u4],
                                  gsem[u4]).wait()
            pltpu.async_copy(rows[u4], acc.at[db.at[u8]], ssem[u4], add=True)

        def body(k, carry):
            j = pl.multiple_of(k * 8, 8)
            for u in range(8):
                step(j + u, u % 4, u)
            return carry

        step(0, 0, 0, first=True)
        step(1, 1, 1)
        step(2, 2, 2)
        step(3, 3, 3)
        step(4, 0, 4)
        step(5, 1, 5)
        step(6, 2, 6)
        step(7, 3, 7)
        nfull = _NCH // 8 - 2
        lax.fori_loop(1, nfull + 1, body, 0)
        for j in range((nfull + 1) * 8, _NCH):
            step(j, j % 4, j % 8)
        for j in range(_NCH - 4, _NCH):
            scat_wait(j, j % 4)
        plsc.subcore_barrier()
        pltpu.sync_copy(acc.at[pl.ds(s * _RPT, _RPT)],
                        out_hbm.at[c, pl.ds(s * _RPT, _RPT)])

    return sc_degree, sc_propagate


def _tc_layer1(x, W1, degp):
    def body(x_ref, w_ref, degp_ref, scaled_ref, dinv_ref):
        dp = degp_ref[...]
        deg = dp[0, :N] + dp[1, :N] + 1.0
        dinv = lax.rsqrt(deg)
        dinv_ref[...] = dinv
        hw = jnp.dot(x_ref[...], w_ref[...], preferred_element_type=jnp.float32)
        scaled_ref[...] = hw * dinv

    return pl.pallas_call(
        body,
        out_shape=(jax.ShapeDtypeStruct((N, D), jnp.float32),
                   jax.ShapeDtypeStruct((N, 1), jnp.float32)),
    )(x, W1, degp)


def _tc_mid(tp, scaled_prev, dinv, b_prev, W):
    def body(tp_ref, sc_ref, dinv_ref, b_ref, w_ref, out_ref):
        tp_ = tp_ref[...]
        t = tp_[0, :N] + tp_[1, :N]
        dinv_ = dinv_ref[...]
        h = jnp.maximum(dinv_ * (t + sc_ref[...]) + b_ref[...], 0.0)
        out_ref[...] = jnp.dot(h, w_ref[...],
                               preferred_element_type=jnp.float32) * dinv_

    return pl.pallas_call(
        body,
        out_shape=jax.ShapeDtypeStruct((N, D), jnp.float32),
    )(tp, scaled_prev, dinv, b_prev, W)


def _tc_final(tp, scaled_prev, dinv, b_prev, batch2d,
              Wm1, bm1, Wm2, bm2, Wh1, bh1, Wh2, bh2):
    def body(tp_ref, sc_ref, dinv_ref, b_ref, batch_ref,
             wm1_ref, bm1_ref, wm2_ref, bm2_ref,
             wh1_ref, bh1_ref, wh2_ref, bh2_ref, out_ref):
        tp_ = tp_ref[...]
        t = tp_[0, :N] + tp_[1, :N]
        h = dinv_ref[...] * (t + sc_ref[...]) + b_ref[...]
        gids = lax.broadcasted_iota(jnp.int32, (N, G), 1)
        onehot = (batch_ref[...] == gids).astype(jnp.float32)
        dn = (((0,), (0,)), ((), ()))
        sums = lax.dot_general(onehot, h, dn,
                               preferred_element_type=jnp.float32)
        counts = lax.dot_general(onehot, jnp.ones((N, 1), jnp.float32), dn,
                                 preferred_element_type=jnp.float32)
        pooled = sums / jnp.maximum(counts, 1.0)
        z = jnp.maximum(jnp.dot(pooled, wm1_ref[...],
                                preferred_element_type=jnp.float32)
                        + bm1_ref[...], 0.0)
        z = jnp.maximum(jnp.dot(z, wm2_ref[...],
                                preferred_element_type=jnp.float32)
                        + bm2_ref[...], 0.0)
        r = jnp.maximum(jnp.dot(z, wh1_ref[...],
                                preferred_element_type=jnp.float32)
                        + bh1_ref[...], 0.0)
        out_ref[...] = jnp.dot(r, wh2_ref[...],
                               preferred_element_type=jnp.float32) + bh2_ref[...]

    return pl.pallas_call(
        body,
        out_shape=jax.ShapeDtypeStruct((G, D), jnp.float32),
    )(tp, scaled_prev, dinv, b_prev, batch2d,
      Wm1, bm1, Wm2, bm2, Wh1, bh1, Wh2, bh2)


def kernel(x, edge_index, batch, W1, b1, W2, b2, W3, b3,
           Wm1, bm1, Wm2, bm2, Wh1, bh1, Wh2, bh2):
    sc_degree, sc_propagate = _sc_kernels()
    ntile = _NCORE * _NSUB
    ereal = E // ntile
    epad = _EPT - ereal
    pad = jnp.arange(epad, dtype=jnp.int32).reshape(1, epad)
    src2 = edge_index[0].astype(jnp.int32).reshape(ntile, ereal)
    dst2 = edge_index[1].astype(jnp.int32).reshape(ntile, ereal)
    src = jnp.concatenate([src2, jnp.broadcast_to(pad, (ntile, epad))], axis=1)
    dst = jnp.concatenate([dst2, jnp.broadcast_to(pad + N, (ntile, epad))],
                          axis=1)
    dst2d = dst.reshape(_EPAD // _KC, _KC)
    src = src.reshape(_EPAD)
    dst = dst.reshape(_EPAD)
    z1 = jnp.zeros((NP,), jnp.float32)
    z2 = jnp.zeros((NP, D), jnp.float32)

    degp = sc_degree(dst2d, z1).reshape(_NCORE, NP, 1)
    scaled1, dinv = _tc_layer1(x, W1, degp)
    t1 = sc_propagate(scaled1, src, dst, z2)
    scaled2 = _tc_mid(t1, scaled1, dinv, b1.reshape(1, D), W2)
    t2 = sc_propagate(scaled2, src, dst, z2)
    scaled3 = _tc_mid(t2, scaled2, dinv, b2.reshape(1, D), W3)
    t3 = sc_propagate(scaled3, src, dst, z2)
    return _tc_final(t3, scaled3, dinv, b3.reshape(1, D),
                     batch.astype(jnp.int32).reshape(N, 1),
                     Wm1, bm1.reshape(1, -1), Wm2, bm2.reshape(1, -1),
                     Wh1, bh1.reshape(1, -1), Wh2, bh2.reshape(1, -1))


# speedup vs baseline: 1.0030x; 1.0030x over previous
import functools

import jax
import jax.numpy as jnp
from jax import lax
from jax.experimental import pallas as pl
from jax.experimental.pallas import tpu as pltpu
from jax.experimental.pallas import tpu_sc as plsc

N = 10000
NP = 10240
E = 320000
G = 256
D = 128

_KC = 64
_NSUB = 16
_NCORE = 2
_NCH = 160
_EPT = _NCH * _KC
_EPAD = _EPT * _NCORE * _NSUB
_RPT = NP // _NSUB


@functools.lru_cache(maxsize=None)
def _sc_kernels():
    mesh = plsc.VectorSubcoreMesh(core_axis_name="c", subcore_axis_name="s")

    @functools.partial(
        pl.kernel,
        mesh=mesh,
        out_type=jax.ShapeDtypeStruct((_NCORE, NP), jnp.float32),
        scratch_types=[
            pltpu.VMEM((_NCH, _KC), jnp.int32),
            pltpu.VMEM((_KC,), jnp.float32),
            pltpu.VMEM_SHARED((NP,), jnp.float32),
            pltpu.SemaphoreType.DMA,
        ],
    )
    def sc_degree(dst_hbm, zeros_hbm, out_hbm, di_v, ones_v, acc, sem):
        c = lax.axis_index("c")
        s = lax.axis_index("s")
        for i in range(_KC // 16):
            ones_v[pl.ds(i * 16, 16)] = jnp.ones((16,), jnp.float32)

        row0 = (c * _NSUB + s) * _NCH
        pltpu.sync_copy(dst_hbm.at[pl.ds(row0, _NCH)], di_v)
        pltpu.sync_copy(zeros_hbm.at[pl.ds(s * _RPT, _RPT)],
                        acc.at[pl.ds(s * _RPT, _RPT)])
        plsc.subcore_barrier()

        def body(k, carry):
            j = pl.multiple_of(k * 8, 8)
            for u in range(8):
                pltpu.async_copy(ones_v, acc.at[di_v.at[j + u]], sem, add=True)
            for u in range(8):
                pltpu.make_async_copy(ones_v, acc.at[di_v.at[j + u]], sem).wait()
            return carry

        lax.fori_loop(0, _NCH // 8, body, 0)
        plsc.subcore_barrier()
        pltpu.sync_copy(acc.at[pl.ds(s * _RPT, _RPT)],
                        out_hbm.at[c, pl.ds(s * _RPT, _RPT)])

    @functools.partial(
        pl.kernel,
        mesh=mesh,
        out_type=jax.ShapeDtypeStruct((_NCORE, NP, D), jnp.float32),
        scratch_types=[
            pltpu.VMEM((_KC, D), jnp.float32),
            pltpu.VMEM((_KC, D), jnp.float32),
            pltpu.VMEM((_KC, D), jnp.float32),
            pltpu.VMEM((_KC, D), jnp.float32),
            pltpu.VMEM((8, _KC), jnp.int32),
            pltpu.VMEM((8, _KC), jnp.int32),
            pltpu.SemaphoreType.DMA,
            pltpu.SemaphoreType.DMA,
            pltpu.SemaphoreType.DMA,
            pltpu.SemaphoreType.DMA,
            pltpu.SemaphoreType.DMA,
            pltpu.SemaphoreType.DMA,
            pltpu.SemaphoreType.DMA,
            pltpu.SemaphoreType.DMA,
            pltpu.SemaphoreType.DMA,
            pltpu.SemaphoreType.DMA,
            pltpu.SemaphoreType.DMA,
            pltpu.SemaphoreType.DMA,
            pltpu.VMEM_SHARED((NP, D), jnp.float32),
        ],
    )
    def sc_propagate(table_hbm, src_hbm, dst_hbm, zeros_hbm, out_hbm,
                     r0, r1, r2, r3, sb, db,
                     i0, i1, i2, i3, i4, i5, i6, i7, g0, g1, g2, g3, acc):
        c = lax.axis_index("c")
        s = lax.axis_index("s")
        rows = (r0, r1, r2, r3)
        isem = (i0, i1, i2, i3, i4, i5, i6, i7)
        gsem = (g0, g1, g2, g3)

        base = (c * _NSUB + s) * _EPT

        def idx_load(j, u6):
            off = pl.multiple_of(base + j * _KC, 8)
            pltpu.async_copy(src_hbm.at[pl.ds(off, _KC)], sb.at[u6], isem[u6])
            pltpu.async_copy(dst_hbm.at[pl.ds(off, _KC)], db.at[u6], isem[u6])

        def idx_wait(u6):
            pltpu.make_async_copy(src_hbm.at[pl.ds(0, _KC)], sb.at[u6],
                                  isem[u6]).wait()
            pltpu.make_async_copy(dst_hbm.at[pl.ds(0, _KC)], db.at[u6],
                                  isem[u6]).wait()

        for m in range(8):
            idx_load(m, m)
        for m in range(3):
            idx_wait(m)
            pltpu.async_copy(table_hbm.at[sb.at[m]], rows[m], gsem[m])
        pltpu.sync_copy(zeros_hbm.at[pl.ds(s * _RPT, _RPT)],
                        acc.at[pl.ds(s * _RPT, _RPT)])
        plsc.subcore_barrier()

        def step(j, u4, u8):
            @pl.when(j + 3 < _NCH)
            def _launch():
                idx_wait((u8 + 3) % 8)
                pltpu.async_copy(table_hbm.at[sb.at[(u8 + 3) % 8]],
                                 rows[(u4 + 3) % 4], gsem[(u4 + 3) % 4])

            pltpu.make_async_copy(table_hbm.at[sb.at[u8]], rows[u4],
                                  gsem[u4]).wait()
            pltpu.sync_copy(rows[u4], acc.at[db.at[u8]], add=True)

            @pl.when(j + 8 < _NCH)
            def _next_idx():
                idx_load(j + 8, u8)

        def body(k, carry):
            j = pl.multiple_of(k * 8, 8)
            for u in range(8):
                step(j + u, u % 4, u)
            return carry

        nfull = _NCH // 8 - 1
        lax.fori_loop(0, nfull, body, 0)
        for j in range(nfull * 8, _NCH):
            step(j, j % 4, j % 8)
        plsc.subcore_barrier()
        pltpu.sync_copy(acc.at[pl.ds(s * _RPT, _RPT)],
                        out_hbm.at[c, pl.ds(s * _RPT, _RPT)])

    return sc_degree, sc_propagate


def _tc_layer1(x, W1, degp):
    def body(x_ref, w_ref, degp_ref, scaled_ref, dinv_ref):
        dp = degp_ref[...]
        deg = dp[0, :N] + dp[1, :N] + 1.0
        dinv = lax.rsqrt(deg)
        dinv_ref[...] = dinv
        hw = jnp.dot(x_ref[...], w_ref[...], preferred_element_type=jnp.float32)
        scaled_ref[...] = hw * dinv

    return pl.pallas_call(
        body,
        out_shape=(jax.ShapeDtypeStruct((N, D), jnp.float32),
                   jax.ShapeDtypeStruct((N, 1), jnp.float32)),
    )(x, W1, degp)


def _tc_mid(tp, scaled_prev, dinv, b_prev, W):
    def body(tp_ref, sc_ref, dinv_ref, b_ref, w_ref, out_ref):
        tp_ = tp_ref[...]
        t = tp_[0, :N] + tp_[1, :N]
        dinv_ = dinv_ref[...]
        h = jnp.maximum(dinv_ * (t + sc_ref[...]) + b_ref[...], 0.0)
        out_ref[...] = jnp.dot(h, w_ref[...],
                               preferred_element_type=jnp.float32) * dinv_

    return pl.pallas_call(
        body,
        out_shape=jax.ShapeDtypeStruct((N, D), jnp.float32),
    )(tp, scaled_prev, dinv, b_prev, W)


def _tc_final(tp, scaled_prev, dinv, b_prev, batch2d,
              Wm1, bm1, Wm2, bm2, Wh1, bh1, Wh2, bh2):
    def body(tp_ref, sc_ref, dinv_ref, b_ref, batch_ref,
             wm1_ref, bm1_ref, wm2_ref, bm2_ref,
             wh1_ref, bh1_ref, wh2_ref, bh2_ref, out_ref):
        tp_ = tp_ref[...]
        t = tp_[0, :N] + tp_[1, :N]
        h = dinv_ref[...] * (t + sc_ref[...]) + b_ref[...]
        gids = lax.broadcasted_iota(jnp.int32, (N, G), 1)
        onehot = (batch_ref[...] == gids).astype(jnp.float32)
        dn = (((0,), (0,)), ((), ()))
        sums = lax.dot_general(onehot, h, dn,
                               preferred_element_type=jnp.float32)
        counts = lax.dot_general(onehot, jnp.ones((N, 1), jnp.float32), dn,
                                 preferred_element_type=jnp.float32)
        pooled = sums / jnp.maximum(counts, 1.0)
        z = jnp.maximum(jnp.dot(pooled, wm1_ref[...],
                                preferred_element_type=jnp.float32)
                        + bm1_ref[...], 0.0)
        z = jnp.maximum(jnp.dot(z, wm2_ref[...],
                                preferred_element_type=jnp.float32)
                        + bm2_ref[...], 0.0)
        r = jnp.maximum(jnp.dot(z, wh1_ref[...],
                                preferred_element_type=jnp.float32)
                        + bh1_ref[...], 0.0)
        out_ref[...] = jnp.dot(r, wh2_ref[...],
                               preferred_element_type=jnp.float32) + bh2_ref[...]

    return pl.pallas_call(
        body,
        out_shape=jax.ShapeDtypeStruct((G, D), jnp.float32),
    )(tp, scaled_prev, dinv, b_prev, batch2d,
      Wm1, bm1, Wm2, bm2, Wh1, bh1, Wh2, bh2)


def kernel(x, edge_index, batch, W1, b1, W2, b2, W3, b3,
           Wm1, bm1, Wm2, bm2, Wh1, bh1, Wh2, bh2):
    sc_degree, sc_propagate = _sc_kernels()
    ntile = _NCORE * _NSUB
    ereal = E // ntile
    epad = _EPT - ereal
    pad = jnp.arange(epad, dtype=jnp.int32).reshape(1, epad)
    src2 = edge_index[0].astype(jnp.int32).reshape(ntile, ereal)
    dst2 = edge_index[1].astype(jnp.int32).reshape(ntile, ereal)
    src = jnp.concatenate([src2, jnp.broadcast_to(pad, (ntile, epad))], axis=1)
    dst = jnp.concatenate([dst2, jnp.broadcast_to(pad + N, (ntile, epad))],
                          axis=1)
    dst2d = dst.reshape(_EPAD // _KC, _KC)
    src = src.reshape(_EPAD)
    dst = dst.reshape(_EPAD)
    z1 = jnp.zeros((NP,), jnp.float32)
    z2 = jnp.zeros((NP, D), jnp.float32)

    degp = sc_degree(dst2d, z1).reshape(_NCORE, NP, 1)
    scaled1, dinv = _tc_layer1(x, W1, degp)
    t1 = sc_propagate(scaled1, src, dst, z2)
    scaled2 = _tc_mid(t1, scaled1, dinv, b1.reshape(1, D), W2)
    t2 = sc_propagate(scaled2, src, dst, z2)
    scaled3 = _tc_mid(t2, scaled2, dinv, b2.reshape(1, D), W3)
    t3 = sc_propagate(scaled3, src, dst, z2)
    return _tc_final(t3, scaled3, dinv, b3.reshape(1, D),
                     batch.astype(jnp.int32).reshape(N, 1),
                     Wm1, bm1.reshape(1, -1), Wm2, bm2.reshape(1, -1),
                     Wh1, bh1.reshape(1, -1), Wh2, bh2.reshape(1, -1))
